# baseline (speedup 1.0000x reference)
import functools

import jax
import jax.numpy as jnp
from jax import lax
from jax.experimental import pallas as pl
from jax.experimental.pallas import tpu as pltpu
from jax.experimental.pallas import tpu_sc as plsc

N = 100000
F = 128
D = 32
NC = 2
NS = 16
NW = NC * NS


@functools.lru_cache(maxsize=None)
def _make_gather_sum(s_fan: int, chunk: int):
    n_chunks = N // chunk
    assert n_chunks * chunk == N
    k_rows = chunk * s_fan
    assert (chunk * D) % 8 == 0
    j_steps = (n_chunks + NW - 1) // NW

    mesh = plsc.VectorSubcoreMesh(
        core_axis_name="c", subcore_axis_name="s",
        num_cores=NC, num_subcores=NS)

    @functools.partial(
        pl.kernel,
        mesh=mesh,
        out_type=jax.ShapeDtypeStruct((N * D,), jnp.float32),
        scratch_types=[
            pltpu.VMEM((2, k_rows), jnp.int32),
            pltpu.VMEM((2, k_rows, D), jnp.float32),
            pltpu.VMEM((2, chunk * D), jnp.float32),
            pltpu.SemaphoreType.DMA,
            pltpu.SemaphoreType.DMA,
            pltpu.SemaphoreType.DMA,
            pltpu.SemaphoreType.DMA,
            pltpu.SemaphoreType.DMA,
            pltpu.SemaphoreType.DMA,
        ],
        compiler_params=pltpu.CompilerParams(use_tc_tiling_on_sc=False),
    )
    def gsum(table_hbm, nidx_hbm, out_hbm, idx_v, rows_v, out_v,
             si0, si1, sg0, sg1, so0, so1):
        wid = lax.axis_index("s") * NC + lax.axis_index("c")
        sis = (si0, si1)
        sgs = (sg0, sg1)
        sos = (so0, so1)

        def cid_of(j):
            return wid + j * NW

        def idx_cp(j, b):
            return pltpu.make_async_copy(
                nidx_hbm.at[cid_of(j)], idx_v.at[b], sis[b])

        def gat_cp(b):
            return pltpu.make_async_copy(
                table_hbm.at[idx_v.at[b]], rows_v.at[b], sgs[b])

        def out_cp(j, b):
            ooff = pl.multiple_of(cid_of(j) * (chunk * D), 8)
            return pltpu.make_async_copy(
                out_v.at[b], out_hbm.at[pl.ds(ooff, chunk * D)], sos[b])

        def when_active(j, fn):
            @pl.when(cid_of(j) < n_chunks)
            def _():
                fn()

        when_active(0, lambda: idx_cp(0, 0).start())
        when_active(1, lambda: idx_cp(1, 1).start())
        when_active(0, lambda: idx_cp(0, 0).wait())
        when_active(0, lambda: gat_cp(0).start())

        def accum(j, b):
            @pl.when(cid_of(j) < n_chunks)
            def _():
                def node(c, carry2):
                    r = c * s_fan
                    a0 = rows_v[b, r, pl.ds(0, 16)]
                    a1 = rows_v[b, r, pl.ds(16, 16)]
                    for s in range(1, s_fan):
                        a0 = a0 + rows_v[b, r + s, pl.ds(0, 16)]
                        a1 = a1 + rows_v[b, r + s, pl.ds(16, 16)]
                    o = pl.multiple_of(c * D, 8)
                    out_v[b, pl.ds(o, 16)] = a0
                    out_v[b, pl.ds(o + 16, 16)] = a1
                    return carry2

                lax.fori_loop(0, chunk, node, 0)
                out_cp(j, b).start()

        def step(jj, carry):
            for b in (0, 1):
                j = jj * 2 + b
                when_active(j, lambda: gat_cp(b).wait())
                when_active(j + 2, lambda: idx_cp(j + 2, b).start())
                when_active(j + 1, lambda: idx_cp(j + 1, 1 - b).wait())
                when_active(j + 1, lambda: gat_cp(1 - b).start())

                @pl.when(jnp.logical_and(jj >= 1, cid_of(j - 2) < n_chunks))
                def _():
                    out_cp(j - 2, b).wait()

                accum(j, b)
            return carry

        lax.fori_loop(0, (j_steps + 1) // 2, step, 0)

        jm = 2 * ((j_steps + 1) // 2)
        for t in (0, 1):
            @pl.when(cid_of(jm - 2 + t) < n_chunks)
            def _(t=t):
                out_cp(jm - 2 + t, t).wait()

    return gsum


NP = N // 4
DP = 4 * D
_PROWS = 1000
_MROWS = 500


def _map_body(x_ref, w_ref, o_ref):
    o_ref[...] = jnp.dot(x_ref[...], w_ref[...],
                         preferred_element_type=jnp.float32)


def _feature_map(features, wt):
    return pl.pallas_call(
        _map_body,
        grid=(NP // _MROWS,),
        in_specs=[pl.BlockSpec((4 * _MROWS, F), lambda i: (i, 0)),
                  pl.BlockSpec((F, D), lambda i: (0, 0))],
        out_specs=pl.BlockSpec((4 * _MROWS, D), lambda i: (i, 0)),
        out_shape=jax.ShapeDtypeStruct((N, D), jnp.float32),
    )(features, wt)


def _update_body(inv, e_ref, a_ref, w_ref, s_ref, o_ref):
    x = (e_ref[...] + a_ref[...]) * inv
    h = jnp.dot(x, w_ref[...], preferred_element_type=jnp.float32)
    h = jnp.maximum(h, 0.0)
    n2 = jnp.dot(h * h, s_ref[...], preferred_element_type=jnp.float32)
    o_ref[...] = h / jnp.maximum(jnp.sqrt(n2), 1e-12)


def _sage_update_packed(embp, nbrp, w4, s4, inv):
    return pl.pallas_call(
        functools.partial(_update_body, inv),
        grid=(NP // _PROWS,),
        in_specs=[pl.BlockSpec((_PROWS, DP), lambda i: (i, 0)),
                  pl.BlockSpec((_PROWS, DP), lambda i: (i, 0)),
                  pl.BlockSpec((DP, DP), lambda i: (0, 0)),
                  pl.BlockSpec((DP, DP), lambda i: (0, 0))],
        out_specs=pl.BlockSpec((_PROWS, DP), lambda i: (i, 0)),
        out_shape=jax.ShapeDtypeStruct((NP, DP), jnp.float32),
    )(embp, nbrp, w4, s4)


def _bdiag4(w):
    z = jnp.zeros_like(w)
    return jnp.block([[w, z, z, z], [z, w, z, z], [z, z, w, z], [z, z, z, w]])


def kernel(features, W_map, W_agg1, W_agg2, neigh1, neigh2):
    n1 = neigh1.astype(jnp.int32).reshape(N // 40, 40 * 25)
    n2 = neigh2.astype(jnp.int32).reshape(N // 160, 160 * 10)
    w4a1 = _bdiag4(W_agg1.T)
    w4a2 = _bdiag4(W_agg2.T)
    s4 = _bdiag4(jnp.ones((D, D), jnp.float32))

    e0f = _feature_map(features, W_map.T).reshape(N * D)
    e0f = lax.optimization_barrier(e0f)
    emb0p = e0f.reshape(NP, DP)
    s1p = _make_gather_sum(25, 40)(e0f.reshape(N, D), n1).reshape(NP, DP)
    emb1p = _sage_update_packed(emb0p, s1p, w4a1, s4, 1.0 / 26.0)
    s2p = _make_gather_sum(10, 160)(emb1p.reshape(N, D), n2).reshape(NP, DP)
    emb2p = _sage_update_packed(emb1p, s2p, w4a2, s4, 1.0 / 11.0)
    return emb2p.reshape(N, D)


# speedup vs baseline: 1.0317x; 1.0317x over previous
import functools

import jax
import jax.numpy as jnp
from jax import lax
from jax.experimental import pallas as pl
from jax.experimental.pallas import tpu as pltpu
from jax.experimental.pallas import tpu_sc as plsc

N = 100000
F = 128
D = 32
NC = 2
NS = 16
NW = NC * NS


@functools.lru_cache(maxsize=None)
def _make_gather_sum(s_fan: int, chunk: int):
    n_chunks = N // chunk
    assert n_chunks * chunk == N
    k_rows = chunk * s_fan
    assert (chunk * D) % 8 == 0
    j_steps = (n_chunks + NW - 1) // NW

    mesh = plsc.VectorSubcoreMesh(
        core_axis_name="c", subcore_axis_name="s",
        num_cores=NC, num_subcores=NS)

    @functools.partial(
        pl.kernel,
        mesh=mesh,
        out_type=jax.ShapeDtypeStruct((N * D,), jnp.float32),
        scratch_types=[
            pltpu.VMEM((2, k_rows), jnp.int32),
            pltpu.VMEM((2, k_rows, D), jnp.float32),
            pltpu.VMEM((2, chunk * D), jnp.float32),
            pltpu.SemaphoreType.DMA,
            pltpu.SemaphoreType.DMA,
            pltpu.SemaphoreType.DMA,
            pltpu.SemaphoreType.DMA,
            pltpu.SemaphoreType.DMA,
            pltpu.SemaphoreType.DMA,
        ],
        compiler_params=pltpu.CompilerParams(use_tc_tiling_on_sc=False),
    )
    def gsum(table_hbm, nidx_hbm, out_hbm, idx_v, rows_v, out_v,
             si0, si1, sg0, sg1, so0, so1):
        wid = lax.axis_index("s") * NC + lax.axis_index("c")
        sis = (si0, si1)
        sgs = (sg0, sg1)
        sos = (so0, so1)

        def cid_of(j):
            return wid + j * NW

        def idx_cp(j, b):
            return pltpu.make_async_copy(
                nidx_hbm.at[cid_of(j)], idx_v.at[b], sis[b])

        def gat_cp(b):
            return pltpu.make_async_copy(
                table_hbm.at[idx_v.at[b]], rows_v.at[b], sgs[b])

        def out_cp(j, b):
            ooff = pl.multiple_of(cid_of(j) * (chunk * D), 8)
            return pltpu.make_async_copy(
                out_v.at[b], out_hbm.at[pl.ds(ooff, chunk * D)], sos[b])

        def when_active(j, fn):
            @pl.when(cid_of(j) < n_chunks)
            def _():
                fn()

        when_active(0, lambda: idx_cp(0, 0).start())
        when_active(1, lambda: idx_cp(1, 1).start())
        when_active(0, lambda: idx_cp(0, 0).wait())
        when_active(0, lambda: gat_cp(0).start())

        def accum(j, b):
            @pl.when(cid_of(j) < n_chunks)
            def _():
                @functools.partial(plsc.parallel_loop, 0, chunk, unroll=2)
                def node(c):
                    r = c * s_fan
                    a0 = rows_v[b, r, pl.ds(0, 16)]
                    a1 = rows_v[b, r, pl.ds(16, 16)]
                    for s in range(1, s_fan):
                        a0 = a0 + rows_v[b, r + s, pl.ds(0, 16)]
                        a1 = a1 + rows_v[b, r + s, pl.ds(16, 16)]
                    o = pl.multiple_of(c * D, 8)
                    out_v[b, pl.ds(o, 16)] = a0
                    out_v[b, pl.ds(o + 16, 16)] = a1

                out_cp(j, b).start()

        def step(jj, carry):
            for b in (0, 1):
                j = jj * 2 + b
                when_active(j, lambda: gat_cp(b).wait())
                when_active(j + 2, lambda: idx_cp(j + 2, b).start())
                when_active(j + 1, lambda: idx_cp(j + 1, 1 - b).wait())
                when_active(j + 1, lambda: gat_cp(1 - b).start())

                @pl.when(jnp.logical_and(jj >= 1, cid_of(j - 2) < n_chunks))
                def _():
                    out_cp(j - 2, b).wait()

                accum(j, b)
            return carry

        lax.fori_loop(0, (j_steps + 1) // 2, step, 0)

        jm = 2 * ((j_steps + 1) // 2)
        for t in (0, 1):
            @pl.when(cid_of(jm - 2 + t) < n_chunks)
            def _(t=t):
                out_cp(jm - 2 + t, t).wait()

    return gsum


NP = N // 4
DP = 4 * D
_PROWS = 1000
_MROWS = 1000


def _map_body(x_ref, w_ref, o_ref):
    o_ref[...] = jnp.dot(x_ref[...], w_ref[...],
                         preferred_element_type=jnp.float32)


def _feature_map(features, wt):
    return pl.pallas_call(
        _map_body,
        grid=(NP // _MROWS,),
        in_specs=[pl.BlockSpec((4 * _MROWS, F), lambda i: (i, 0)),
                  pl.BlockSpec((F, D), lambda i: (0, 0))],
        out_specs=pl.BlockSpec((4 * _MROWS, D), lambda i: (i, 0)),
        out_shape=jax.ShapeDtypeStruct((N, D), jnp.float32),
    )(features, wt)


def _update_body(inv, e_ref, a_ref, w_ref, s_ref, o_ref):
    x = (e_ref[...] + a_ref[...]) * inv
    h = jnp.dot(x, w_ref[...], preferred_element_type=jnp.float32)
    h = jnp.maximum(h, 0.0)
    n2 = jnp.dot(h * h, s_ref[...], preferred_element_type=jnp.float32)
    o_ref[...] = h / jnp.maximum(jnp.sqrt(n2), 1e-12)


def _sage_update_packed(embp, nbrp, w4, s4, inv):
    return pl.pallas_call(
        functools.partial(_update_body, inv),
        grid=(NP // _PROWS,),
        in_specs=[pl.BlockSpec((_PROWS, DP), lambda i: (i, 0)),
                  pl.BlockSpec((_PROWS, DP), lambda i: (i, 0)),
                  pl.BlockSpec((DP, DP), lambda i: (0, 0)),
                  pl.BlockSpec((DP, DP), lambda i: (0, 0))],
        out_specs=pl.BlockSpec((_PROWS, DP), lambda i: (i, 0)),
        out_shape=jax.ShapeDtypeStruct((NP, DP), jnp.float32),
    )(embp, nbrp, w4, s4)


def _bdiag4(w):
    z = jnp.zeros_like(w)
    return jnp.block([[w, z, z, z], [z, w, z, z], [z, z, w, z], [z, z, z, w]])


def kernel(features, W_map, W_agg1, W_agg2, neigh1, neigh2):
    n1 = neigh1.astype(jnp.int32).reshape(N // 40, 40 * 25)
    n2 = neigh2.astype(jnp.int32).reshape(N // 160, 160 * 10)
    w4a1 = _bdiag4(W_agg1.T)
    w4a2 = _bdiag4(W_agg2.T)
    s4 = _bdiag4(jnp.ones((D, D), jnp.float32))

    e0f = _feature_map(features, W_map.T).reshape(N * D)
    e0f = lax.optimization_barrier(e0f)
    emb0p = e0f.reshape(NP, DP)
    s1p = _make_gather_sum(25, 40)(e0f.reshape(N, D), n1).reshape(NP, DP)
    emb1p = _sage_update_packed(emb0p, s1p, w4a1, s4, 1.0 / 26.0)
    s2p = _make_gather_sum(10, 160)(emb1p.reshape(N, D), n2).reshape(NP, DP)
    emb2p = _sage_update_packed(emb1p, s2p, w4a2, s4, 1.0 / 11.0)
    return emb2p.reshape(N, D)


# speedup vs baseline: 1.0411x; 1.0092x over previous
import functools

import jax
import jax.numpy as jnp
from jax import lax
from jax.experimental import pallas as pl
from jax.experimental.pallas import tpu as pltpu
from jax.experimental.pallas import tpu_sc as plsc

N = 100000
F = 128
D = 32
NC = 2
NS = 16
NW = NC * NS


@functools.lru_cache(maxsize=None)
def _make_gather_sum(s_fan: int, chunk: int):
    n_chunks = N // chunk
    assert n_chunks * chunk == N
    k_rows = chunk * s_fan
    assert (chunk * D) % 8 == 0
    j_steps = (n_chunks + NW - 1) // NW

    mesh = plsc.VectorSubcoreMesh(
        core_axis_name="c", subcore_axis_name="s",
        num_cores=NC, num_subcores=NS)

    @functools.partial(
        pl.kernel,
        mesh=mesh,
        out_type=jax.ShapeDtypeStruct((N * D,), jnp.float32),
        scratch_types=[
            pltpu.VMEM((2, k_rows), jnp.int32),
            pltpu.VMEM((2, k_rows, D), jnp.float32),
            pltpu.VMEM((2, chunk * D), jnp.float32),
            pltpu.SemaphoreType.DMA,
            pltpu.SemaphoreType.DMA,
            pltpu.SemaphoreType.DMA,
            pltpu.SemaphoreType.DMA,
            pltpu.SemaphoreType.DMA,
            pltpu.SemaphoreType.DMA,
        ],
        compiler_params=pltpu.CompilerParams(use_tc_tiling_on_sc=False),
    )
    def gsum(table_hbm, nidx_hbm, out_hbm, idx_v, rows_v, out_v,
             si0, si1, sg0, sg1, so0, so1):
        wid = lax.axis_index("s") * NC + lax.axis_index("c")
        sis = (si0, si1)
        sgs = (sg0, sg1)
        sos = (so0, so1)

        def cid_of(j):
            return wid + j * NW

        def idx_cp(j, b):
            return pltpu.make_async_copy(
                nidx_hbm.at[cid_of(j)], idx_v.at[b], sis[b])

        def gat_cp(b):
            return pltpu.make_async_copy(
                table_hbm.at[idx_v.at[b]], rows_v.at[b], sgs[b])

        def out_cp(j, b):
            ooff = pl.multiple_of(cid_of(j) * (chunk * D), 8)
            return pltpu.make_async_copy(
                out_v.at[b], out_hbm.at[pl.ds(ooff, chunk * D)], sos[b])

        def when_active(j, fn):
            @pl.when(cid_of(j) < n_chunks)
            def _():
                fn()

        when_active(0, lambda: idx_cp(0, 0).start())
        when_active(1, lambda: idx_cp(1, 1).start())
        when_active(0, lambda: idx_cp(0, 0).wait())
        when_active(0, lambda: gat_cp(0).start())

        def accum(j, b):
            @pl.when(cid_of(j) < n_chunks)
            def _():
                @functools.partial(plsc.parallel_loop, 0, chunk, unroll=2)
                def node(c):
                    r = c * s_fan
                    a0 = rows_v[b, r, pl.ds(0, 16)]
                    a1 = rows_v[b, r, pl.ds(16, 16)]
                    for s in range(1, s_fan):
                        a0 = a0 + rows_v[b, r + s, pl.ds(0, 16)]
                        a1 = a1 + rows_v[b, r + s, pl.ds(16, 16)]
                    o = pl.multiple_of(c * D, 8)
                    out_v[b, pl.ds(o, 16)] = a0
                    out_v[b, pl.ds(o + 16, 16)] = a1

                out_cp(j, b).start()

        def step(jj, carry):
            for b in (0, 1):
                j = jj * 2 + b
                when_active(j, lambda: gat_cp(b).wait())
                when_active(j + 2, lambda: idx_cp(j + 2, b).start())
                when_active(j + 1, lambda: idx_cp(j + 1, 1 - b).wait())
                when_active(j + 1, lambda: gat_cp(1 - b).start())

                @pl.when(jnp.logical_and(jj >= 1, cid_of(j - 2) < n_chunks))
                def _():
                    out_cp(j - 2, b).wait()

                accum(j, b)
            return carry

        lax.fori_loop(0, (j_steps + 1) // 2, step, 0)

        jm = 2 * ((j_steps + 1) // 2)
        for t in (0, 1):
            @pl.when(cid_of(jm - 2 + t) < n_chunks)
            def _(t=t):
                out_cp(jm - 2 + t, t).wait()

    return gsum


NP = N // 4
DP = 4 * D
_PROWS = 1000
_MROWS = 2500


def _map_body(x_ref, w_ref, o_ref):
    o_ref[...] = jnp.dot(x_ref[...], w_ref[...],
                         preferred_element_type=jnp.float32)


def _feature_map(features, wt):
    return pl.pallas_call(
        _map_body,
        grid=(NP // _MROWS,),
        in_specs=[pl.BlockSpec((4 * _MROWS, F), lambda i: (i, 0)),
                  pl.BlockSpec((F, D), lambda i: (0, 0))],
        out_specs=pl.BlockSpec((4 * _MROWS, D), lambda i: (i, 0)),
        out_shape=jax.ShapeDtypeStruct((N, D), jnp.float32),
    )(features, wt)


def _update_body(inv, e_ref, a_ref, w_ref, s_ref, o_ref):
    x = (e_ref[...] + a_ref[...]) * inv
    h = jnp.dot(x, w_ref[...], preferred_element_type=jnp.float32)
    h = jnp.maximum(h, 0.0)
    n2 = jnp.dot(h * h, s_ref[...], preferred_element_type=jnp.float32)
    o_ref[...] = h / jnp.maximum(jnp.sqrt(n2), 1e-12)


def _sage_update_packed(embp, nbrp, w4, s4, inv):
    return pl.pallas_call(
        functools.partial(_update_body, inv),
        grid=(NP // _PROWS,),
        in_specs=[pl.BlockSpec((_PROWS, DP), lambda i: (i, 0)),
                  pl.BlockSpec((_PROWS, DP), lambda i: (i, 0)),
                  pl.BlockSpec((DP, DP), lambda i: (0, 0)),
                  pl.BlockSpec((DP, DP), lambda i: (0, 0))],
        out_specs=pl.BlockSpec((_PROWS, DP), lambda i: (i, 0)),
        out_shape=jax.ShapeDtypeStruct((NP, DP), jnp.float32),
    )(embp, nbrp, w4, s4)


def _bdiag4(w):
    z = jnp.zeros_like(w)
    return jnp.block([[w, z, z, z], [z, w, z, z], [z, z, w, z], [z, z, z, w]])


def kernel(features, W_map, W_agg1, W_agg2, neigh1, neigh2):
    n1 = neigh1.astype(jnp.int32).reshape(N // 40, 40 * 25)
    n2 = neigh2.astype(jnp.int32).reshape(N // 160, 160 * 10)
    w4a1 = _bdiag4(W_agg1.T)
    w4a2 = _bdiag4(W_agg2.T)
    s4 = _bdiag4(jnp.ones((D, D), jnp.float32))

    e0f = _feature_map(features, W_map.T).reshape(N * D)
    e0f = lax.optimization_barrier(e0f)
    emb0p = e0f.reshape(NP, DP)
    s1p = _make_gather_sum(25, 40)(e0f.reshape(N, D), n1).reshape(NP, DP)
    emb1p = _sage_update_packed(emb0p, s1p, w4a1, s4, 1.0 / 26.0)
    s2p = _make_gather_sum(10, 160)(emb1p.reshape(N, D), n2).reshape(NP, DP)
    emb2p = _sage_update_packed(emb1p, s2p, w4a2, s4, 1.0 / 11.0)
    return emb2p.reshape(N, D)
